# baseline (speedup 1.0000x reference)
import functools

import jax
import jax.numpy as jnp
from jax import lax
from jax.experimental import pallas as pl
from jax.experimental.pallas import tpu as pltpu
from jax.experimental.pallas import tpu_sc as plsc

_CW = 512
_L = 16
_SUP = 8
_CAP = 2048


@functools.lru_cache(maxsize=None)
def _build(B, V, D):
    info = plsc.get_sparse_core_info()
    NC, NS = info.num_cores, info.num_subcores
    NW = NC * NS
    assert B % _L == 0 and D % _L == 0
    nch = V // _CW
    tailw = V - nch * _CW
    mesh = plsc.VectorSubcoreMesh(core_axis_name="c", subcore_axis_name="s")

    scratch = [
        pltpu.VMEM((B,), jnp.int32),
        pltpu.VMEM((B + _L,), jnp.int32),
        pltpu.VMEM((B + _L,), jnp.int32),
        pltpu.VMEM((_CAP + 2 * _L,), jnp.int32),
        pltpu.VMEM((_CAP + 2 * _L,), jnp.int32),
        pltpu.VMEM((D, _CW), jnp.float32),
        pltpu.VMEM((D, _CW), jnp.float32),
        pltpu.VMEM((_L, D), jnp.float32),
        pltpu.SemaphoreType.DMA,
        pltpu.SemaphoreType.DMA,
        pltpu.SemaphoreType.DMA,
    ]
    if tailw:
        scratch.append(pltpu.VMEM((D, tailw), jnp.float32))

    @functools.partial(
        pl.kernel,
        mesh=mesh,
        out_type=jax.ShapeDtypeStruct((B, D), jnp.float32),
        scratch_types=scratch,
        compiler_params=pltpu.CompilerParams(needs_layout_passes=False),
    )
    def k(idx_hbm, tablet_hbm, *rest):
        if tailw:
            tail_hbm, out_hbm, idx_v, mi_v, mr_v, smi_v, smr_v, buf0, buf1, \
                stage_v, sem0, sem1, sem_out, tailbuf = rest
        else:
            out_hbm, idx_v, mi_v, mr_v, smi_v, smr_v, buf0, buf1, \
                stage_v, sem0, sem1, sem_out = rest
        wid = lax.axis_index("s") * NC + lax.axis_index("c")
        c0 = (wid * nch) // NW
        c1 = ((wid + 1) * nch) // NW
        nlo = c0 * _CW
        nhi = jnp.where(wid == NW - 1, V, c1 * _CW)
        iota16 = lax.iota(jnp.int32, _L)
        sentinel = jnp.full((_L,), V, jnp.int32)

        pltpu.sync_copy(idx_hbm, idx_v)

        def fbody(g, cursor):
            iv = idx_v[pl.ds(g * _L, _L)]
            m = (iv >= nlo) & (iv < nhi)
            pos = plsc.cumsum(jnp.where(m, 1, 0))
            tgt = jnp.where(m, cursor + pos - 1, jnp.int32(B))
            plsc.store_scatter(mi_v, [tgt], iv)
            plsc.store_scatter(mr_v, [tgt], g * _L + iota16)
            return cursor + pos[_L - 1]

        ncand = pl.loop(0, B // _L, init_carry=jnp.int32(0), unroll=4)(fbody)
        mi_v[pl.ds(ncand, _L)] = sentinel
        ng = (ncand + _L - 1) // _L

        def issue(kc, buf, sem):
            pltpu.async_copy(tablet_hbm.at[:, pl.ds(kc * _CW, _CW)], buf, sem)

        def wait(buf, sem):
            pltpu.make_async_copy(
                tablet_hbm.at[:, pl.ds(0, _CW)], buf, sem
            ).wait()

        def recompact(slo, shi):
            def rbody(g, sc):
                civ = mi_v[pl.ds(g * _L, _L)]
                m = (civ >= slo) & (civ < shi)
                pos = plsc.cumsum(jnp.where(m, 1, 0))
                tgt = jnp.minimum(
                    jnp.where(m, sc + pos - 1, jnp.int32(_CAP)),
                    jnp.int32(_CAP),
                )
                plsc.store_scatter(smi_v, [tgt], civ)
                crv = mr_v[pl.ds(g * _L, _L)]
                plsc.store_scatter(smr_v, [tgt], crv)
                return sc + pos[_L - 1]

            sc = pl.loop(0, ng, init_carry=jnp.int32(0))(rbody)
            sc2 = jnp.minimum(sc, jnp.int32(_CAP))
            smi_v[pl.ds(sc2, _L)] = sentinel
            return sc2, (sc > _CAP).astype(jnp.int32)

        def process(clo, chi, buf, milist, mrlist, ngroups):
            def pgbody(g):
                civ = milist[pl.ds(g * _L, _L)]
                m = (civ >= clo) & (civ < chi)

                @pl.when(jnp.any(m))
                def _():
                    crv = mrlist[pl.ds(g * _L, _L)]
                    nen = jnp.int32(0)
                    for j in range(_L):
                        cj = civ[j]
                        okj = (cj >= clo) & (cj < chi)

                        @pl.when(okj)
                        def _():
                            lvec = jnp.full((_L,), cj - clo, jnp.int32)
                            for cb in range(0, D, _L):
                                vals = plsc.load_gather(
                                    buf, [cb + iota16, lvec]
                                )
                                stage_v[j, pl.ds(cb, _L)] = vals
                            pltpu.async_copy(
                                stage_v.at[pl.ds(j, 1)],
                                out_hbm.at[pl.ds(crv[j], 1)],
                                sem_out,
                            )

                        nen = lax.select(okj, nen + 1, nen)

                    def dbody(_):
                        pltpu.make_async_copy(
                            out_hbm.at[pl.ds(0, 1)],
                            stage_v.at[pl.ds(0, 1)],
                            sem_out,
                        ).wait()

                    pl.loop(0, nen)(dbody)

            pl.loop(0, ngroups)(pgbody)

        def proc_either(kc, buf, subn, over):
            clo = kc * _CW
            chi = clo + _CW
            ngs = (subn + _L - 1) // _L

            @pl.when(over == 0)
            def _():
                process(clo, chi, buf, smi_v, smr_v, ngs)

            @pl.when(over != 0)
            def _():
                process(clo, chi, buf, mi_v, mr_v, ng)

        @pl.when(c0 < c1)
        def _():
            issue(c0, buf0, sem0)

        @pl.when(c0 + 1 < c1)
        def _():
            issue(c0 + 1, buf1, sem1)

        def sbody(kc, carry):
            subn, over = carry
            at_super = ((kc - c0) & (_SUP - 1)) == 0

            new = lax.cond(
                at_super,
                lambda: recompact(
                    kc * _CW, jnp.minimum(kc + _SUP, c1) * _CW
                ),
                lambda: (subn, over),
            )
            subn, over = new

            wait(buf0, sem0)
            proc_either(kc, buf0, subn, over)

            @pl.when(kc + 2 < c1)
            def _():
                issue(kc + 2, buf0, sem0)

            @pl.when(kc + 1 < c1)
            def _():
                wait(buf1, sem1)
                proc_either(kc + 1, buf1, subn, over)

                @pl.when(kc + 3 < c1)
                def _():
                    issue(kc + 3, buf1, sem1)

            return (subn, over)

        pl.loop(
            c0, c1, step=2,
            init_carry=(jnp.int32(0), jnp.int32(0)),
        )(sbody)

        if tailw:

            @pl.when(wid == NW - 1)
            def _():
                pltpu.sync_copy(tail_hbm, tailbuf)
                process(nch * _CW, V, tailbuf, mi_v, mr_v, ng)

    return k


def kernel(idx, table):
    (B,) = idx.shape
    V, D = table.shape
    tailw = V % _CW
    args = (idx, table.T)
    if tailw:
        args = args + (table[V - tailw :, :].T,)
    return _build(B, V, D)(*args)


# speedup vs baseline: 1.0377x; 1.0377x over previous
import functools

import jax
import jax.numpy as jnp
from jax import lax
from jax.experimental import pallas as pl
from jax.experimental.pallas import tpu as pltpu
from jax.experimental.pallas import tpu_sc as plsc

_CW = 512
_L = 16
_SUP = 8
_CAP = 2048


@functools.lru_cache(maxsize=None)
def _build(B, V, D):
    info = plsc.get_sparse_core_info()
    NC, NS = info.num_cores, info.num_subcores
    NW = NC * NS
    assert B % _L == 0 and D % _L == 0
    nch = V // _CW
    tailw = V - nch * _CW
    mesh = plsc.VectorSubcoreMesh(core_axis_name="c", subcore_axis_name="s")

    scratch = [
        pltpu.VMEM((B,), jnp.int32),
        pltpu.VMEM((B + _L,), jnp.int32),
        pltpu.VMEM((B + _L,), jnp.int32),
        pltpu.VMEM((_CAP + 2 * _L,), jnp.int32),
        pltpu.VMEM((_CAP + 2 * _L,), jnp.int32),
        pltpu.VMEM((D, _CW), jnp.float32),
        pltpu.VMEM((D, _CW), jnp.float32),
        pltpu.VMEM((_L, D), jnp.float32),
        pltpu.SemaphoreType.DMA,
        pltpu.SemaphoreType.DMA,
        pltpu.SemaphoreType.DMA,
    ]
    if tailw:
        scratch.append(pltpu.VMEM((D, tailw), jnp.float32))

    @functools.partial(
        pl.kernel,
        mesh=mesh,
        out_type=jax.ShapeDtypeStruct((B, D), jnp.float32),
        scratch_types=scratch,
        compiler_params=pltpu.CompilerParams(needs_layout_passes=False),
    )
    def k(idx_hbm, tablet_hbm, *rest):
        if tailw:
            tail_hbm, out_hbm, idx_v, mi_v, mr_v, smi_v, smr_v, buf0, buf1, \
                stage_v, sem0, sem1, sem_out, tailbuf = rest
        else:
            out_hbm, idx_v, mi_v, mr_v, smi_v, smr_v, buf0, buf1, \
                stage_v, sem0, sem1, sem_out = rest
        wid = lax.axis_index("s") * NC + lax.axis_index("c")
        c0 = (wid * nch) // NW
        c1 = ((wid + 1) * nch) // NW
        nlo = c0 * _CW
        nhi = jnp.where(wid == NW - 1, V, c1 * _CW)
        iota16 = lax.iota(jnp.int32, _L)
        sentinel = jnp.full((_L,), V, jnp.int32)

        pltpu.sync_copy(idx_hbm, idx_v)

        def fbody(g, cursor):
            iv = idx_v[pl.ds(g * _L, _L)]
            m = (iv >= nlo) & (iv < nhi)
            pos = plsc.cumsum(jnp.where(m, 1, 0))
            tgt = jnp.where(m, cursor + pos - 1, jnp.int32(B))
            plsc.store_scatter(mi_v, [tgt], iv)
            plsc.store_scatter(mr_v, [tgt], g * _L + iota16)
            return cursor + pos[_L - 1]

        def issue(kc, buf, sem):
            pltpu.async_copy(tablet_hbm.at[:, pl.ds(kc * _CW, _CW)], buf, sem)

        @pl.when(c0 < c1)
        def _():
            issue(c0, buf0, sem0)

        @pl.when(c0 + 1 < c1)
        def _():
            issue(c0 + 1, buf1, sem1)

        ncand = pl.loop(0, B // _L, init_carry=jnp.int32(0))(fbody)
        mi_v[pl.ds(ncand, _L)] = sentinel
        ng = (ncand + _L - 1) // _L

        def wait(buf, sem):
            pltpu.make_async_copy(
                tablet_hbm.at[:, pl.ds(0, _CW)], buf, sem
            ).wait()

        def recompact(slo, shi):
            def rbody(g, sc):
                civ = mi_v[pl.ds(g * _L, _L)]
                m = (civ >= slo) & (civ < shi)
                pos = plsc.cumsum(jnp.where(m, 1, 0))
                tgt = jnp.minimum(
                    jnp.where(m, sc + pos - 1, jnp.int32(_CAP)),
                    jnp.int32(_CAP),
                )
                plsc.store_scatter(smi_v, [tgt], civ)
                crv = mr_v[pl.ds(g * _L, _L)]
                plsc.store_scatter(smr_v, [tgt], crv)
                return sc + pos[_L - 1]

            sc = pl.loop(0, ng, init_carry=jnp.int32(0))(rbody)
            sc2 = jnp.minimum(sc, jnp.int32(_CAP))
            smi_v[pl.ds(sc2, _L)] = sentinel
            return sc2, (sc > _CAP).astype(jnp.int32)

        def process(clo, chi, buf, milist, mrlist, ngroups):
            def pgbody(g):
                civ = milist[pl.ds(g * _L, _L)]
                m = (civ >= clo) & (civ < chi)

                @pl.when(jnp.any(m))
                def _():
                    crv = mrlist[pl.ds(g * _L, _L)]
                    nen = jnp.int32(0)
                    for j in range(_L):
                        cj = civ[j]
                        okj = (cj >= clo) & (cj < chi)

                        @pl.when(okj)
                        def _():
                            lvec = jnp.full((_L,), cj - clo, jnp.int32)
                            for cb in range(0, D, _L):
                                vals = plsc.load_gather(
                                    buf, [cb + iota16, lvec]
                                )
                                stage_v[j, pl.ds(cb, _L)] = vals
                            pltpu.async_copy(
                                stage_v.at[pl.ds(j, 1)],
                                out_hbm.at[pl.ds(crv[j], 1)],
                                sem_out,
                            )

                        nen = lax.select(okj, nen + 1, nen)

                    def dbody(_):
                        pltpu.make_async_copy(
                            out_hbm.at[pl.ds(0, 1)],
                            stage_v.at[pl.ds(0, 1)],
                            sem_out,
                        ).wait()

                    pl.loop(0, nen)(dbody)

            pl.loop(0, ngroups)(pgbody)

        def proc_either(kc, buf, subn, over):
            clo = kc * _CW
            chi = clo + _CW
            ngs = (subn + _L - 1) // _L

            @pl.when(over == 0)
            def _():
                process(clo, chi, buf, smi_v, smr_v, ngs)

            @pl.when(over != 0)
            def _():
                process(clo, chi, buf, mi_v, mr_v, ng)

        def sbody(kc, carry):
            subn, over = carry
            at_super = ((kc - c0) & (_SUP - 1)) == 0

            new = lax.cond(
                at_super,
                lambda: recompact(
                    kc * _CW, jnp.minimum(kc + _SUP, c1) * _CW
                ),
                lambda: (subn, over),
            )
            subn, over = new

            wait(buf0, sem0)
            proc_either(kc, buf0, subn, over)

            @pl.when(kc + 2 < c1)
            def _():
                issue(kc + 2, buf0, sem0)

            @pl.when(kc + 1 < c1)
            def _():
                wait(buf1, sem1)
                proc_either(kc + 1, buf1, subn, over)

                @pl.when(kc + 3 < c1)
                def _():
                    issue(kc + 3, buf1, sem1)

            return (subn, over)

        pl.loop(
            c0, c1, step=2,
            init_carry=(jnp.int32(0), jnp.int32(0)),
        )(sbody)

        if tailw:

            @pl.when(wid == NW - 1)
            def _():
                pltpu.sync_copy(tail_hbm, tailbuf)
                process(nch * _CW, V, tailbuf, mi_v, mr_v, ng)

    return k


def kernel(idx, table):
    (B,) = idx.shape
    V, D = table.shape
    tailw = V % _CW
    args = (idx, table.T)
    if tailw:
        args = args + (table[V - tailw :, :].T,)
    return _build(B, V, D)(*args)
